# baseline (speedup 1.0000x reference)
import functools

import jax
import jax.numpy as jnp
from jax import lax
from jax.experimental import pallas as pl
from jax.experimental.pallas import tpu as pltpu
from jax.experimental.pallas import tpu_sc as plsc

_SMOOTH = 1.0
_NC = 19
_HT = 256
_NS = 512 // _HT
_NB_TC = 6
_SC_B0 = _NB_TC
_NB_SC = 8 - _NB_TC

_P = 512 * 512
_NW = 32
_CHUNK = _P // _NW
_PAIRS = [(b, c) for b in range(_SC_B0, 8) for c in range(_NC)]
_NPAIR = len(_PAIRS)
_RES_SLOTS = 40
_RES_LEN = 3 * _RES_SLOTS * 16


def _dice_tc_body(pred_ref, tgt_ref, out_ref, nom_acc, isum_acc, tsum_acc):
    s = pl.program_id(1)

    @pl.when(s == 0)
    def _init():
        nom_acc[...] = jnp.zeros_like(nom_acc)
        isum_acc[...] = jnp.zeros_like(isum_acc)
        tsum_acc[...] = jnp.zeros_like(tsum_acc)

    tgt = tgt_ref[0]
    for c in range(_NC):
        pred_c = pred_ref[0, c]
        mask = tgt == c
        nom_p = jnp.where(mask, pred_c, 0.0).reshape(_HT // 8, 8, 512).sum(axis=0)
        isum_p = pred_c.reshape(_HT // 8, 8, 512).sum(axis=0)
        tsum_p = jnp.where(mask, 1.0, 0.0).reshape(_HT // 8, 8, 512).sum(axis=0)
        nom_acc[c] += nom_p
        isum_acc[c] += isum_p
        tsum_acc[c] += tsum_p

    @pl.when(s == _NS - 1)
    def _finish():
        nom = jnp.sum(nom_acc[...], axis=(1, 2))
        isum = jnp.sum(isum_acc[...], axis=(1, 2))
        tsum = jnp.sum(tsum_acc[...], axis=(1, 2))
        frac = (2.0 * nom + _SMOOTH) / (isum + tsum + _SMOOTH)
        loss = -jnp.sum(frac) / _NC
        out_ref[0, 0, :] = jnp.full((128,), loss, dtype=jnp.float32)


def _tc_losses(predictions, targets):
    out = pl.pallas_call(
        _dice_tc_body,
        grid=(_NB_TC, _NS),
        in_specs=[
            pl.BlockSpec((1, _NC, _HT, 512), lambda b, s: (b, 0, s, 0)),
            pl.BlockSpec((1, _HT, 512), lambda b, s: (b, s, 0)),
        ],
        out_specs=pl.BlockSpec((1, 1, 128), lambda b, s: (b, 0, 0)),
        out_shape=jax.ShapeDtypeStruct((_NB_TC, 1, 128), jnp.float32),
        scratch_shapes=[
            pltpu.VMEM((_NC, 8, 512), jnp.float32),
            pltpu.VMEM((_NC, 8, 512), jnp.float32),
            pltpu.VMEM((_NC, 8, 512), jnp.float32),
        ],
    )(predictions, targets)
    return out[:, 0, 0]


def _dice_sc_body(pred_hbm, tgt_hbm, out_hbm,
                  tgt_v, pb0, pb1, res_v, semt, sem0, sem1, semo):
    wid = lax.axis_index("s") * 2 + lax.axis_index("c")
    base = wid * _CHUNK

    tw0 = pltpu.async_copy(tgt_hbm.at[pl.ds(_SC_B0 * _P + base, _CHUNK)],
                           tgt_v.at[pl.ds(0, _CHUNK)], semt)
    tw1 = pltpu.async_copy(tgt_hbm.at[pl.ds((_SC_B0 + 1) * _P + base, _CHUNK)],
                           tgt_v.at[pl.ds(_CHUNK, _CHUNK)], semt)

    def issue(j):
        b, c = _PAIRS[j]
        buf, sem = (pb0, sem0) if j % 2 == 0 else (pb1, sem1)
        return pltpu.async_copy(
            pred_hbm.at[pl.ds((b * _NC + c) * _P + base, _CHUNK)], buf, sem)

    handles = [issue(0)]
    tw0.wait()
    tw1.wait()

    zero = jnp.zeros((16,), jnp.float32)
    for j, (b, c) in enumerate(_PAIRS):
        if j + 1 < _NPAIR:
            handles.append(issue(j + 1))
        handles[j].wait()
        buf = pb0 if j % 2 == 0 else pb1
        ti = b - _SC_B0

        def vec_body(v, carry, buf=buf, ti=ti, c=c):
            nom_v, isum_v, tsum_v = carry
            p = buf[pl.ds(v * 16, 16)]
            t = tgt_v[pl.ds(ti * _CHUNK + v * 16, 16)]
            maskf = jnp.where(t == c, 1.0, 0.0)
            return (nom_v + p * maskf, isum_v + p, tsum_v + maskf)

        nom_v, isum_v, tsum_v = lax.fori_loop(
            0, _CHUNK // 16, vec_body, (zero, zero, zero), unroll=8)
        res_v[pl.ds((0 * _RES_SLOTS + j) * 16, 16)] = nom_v
        res_v[pl.ds((1 * _RES_SLOTS + j) * 16, 16)] = isum_v
        res_v[pl.ds((2 * _RES_SLOTS + j) * 16, 16)] = tsum_v

    pltpu.async_copy(res_v, out_hbm.at[pl.ds(wid * _RES_LEN, _RES_LEN)],
                     semo).wait()


@functools.partial(
    pl.kernel,
    out_type=jax.ShapeDtypeStruct((_NW * _RES_LEN,), jnp.float32),
    mesh=plsc.VectorSubcoreMesh(core_axis_name="c", subcore_axis_name="s"),
    scratch_types=[
        pltpu.VMEM((_NB_SC * _CHUNK,), jnp.int32),
        pltpu.VMEM((_CHUNK,), jnp.float32),
        pltpu.VMEM((_CHUNK,), jnp.float32),
        pltpu.VMEM((_RES_LEN,), jnp.float32),
        pltpu.SemaphoreType.DMA,
        pltpu.SemaphoreType.DMA,
        pltpu.SemaphoreType.DMA,
        pltpu.SemaphoreType.DMA,
    ],
)
def _sc_partials(pred_hbm, tgt_hbm, out_hbm, *scratch):
    _dice_sc_body(pred_hbm, tgt_hbm, out_hbm, *scratch)


def kernel(predictions, targets):
    pred_flat = predictions.reshape(8 * _NC * _P)
    tgt_flat = targets.reshape(8 * _P)

    loss_tc = _tc_losses(predictions, targets)
    parts = _sc_partials(pred_flat, tgt_flat)
    parts = parts.reshape(_NW, 3, _RES_SLOTS, 16)

    sums = parts.sum(axis=(0, 3))
    nom = sums[0, :_NPAIR].reshape(_NB_SC, _NC)
    isum = sums[1, :_NPAIR].reshape(_NB_SC, _NC)
    tsum = sums[2, :_NPAIR].reshape(_NB_SC, _NC)
    frac = (2.0 * nom + _SMOOTH) / (isum + tsum + _SMOOTH)
    loss_sc = -jnp.mean(frac, axis=1)

    return jnp.concatenate([loss_tc, loss_sc])


# speedup vs baseline: 2.4586x; 2.4586x over previous
import functools

import jax
import jax.numpy as jnp
from jax import lax
from jax.experimental import pallas as pl
from jax.experimental.pallas import tpu as pltpu
from jax.experimental.pallas import tpu_sc as plsc

_SMOOTH = 1.0
_NC = 19
_HT = 256
_NS = 512 // _HT
_NB_TC = 6
_SC_B0 = _NB_TC
_NB_SC = 8 - _NB_TC

_P = 512 * 512
_NW = 32
_ROWS = 512 // _NW
_CHUNK = _ROWS * 512
_PAIRS = [(b, c) for b in range(_SC_B0, 8) for c in range(_NC)]
_NPAIR = len(_PAIRS)
_RES_SLOTS = 40
_RES_LEN = 3 * _RES_SLOTS * 16


def _dice_tc_body(pred_ref, tgt_ref, out_ref, nom_acc, isum_acc, tsum_acc):
    s = pl.program_id(1)

    @pl.when(s == 0)
    def _init():
        nom_acc[...] = jnp.zeros_like(nom_acc)
        isum_acc[...] = jnp.zeros_like(isum_acc)
        tsum_acc[...] = jnp.zeros_like(tsum_acc)

    tgt = tgt_ref[0]
    for c in range(_NC):
        pred_c = pred_ref[0, c]
        mask = tgt == c
        nom_p = jnp.where(mask, pred_c, 0.0).reshape(_HT // 8, 8, 512).sum(axis=0)
        isum_p = pred_c.reshape(_HT // 8, 8, 512).sum(axis=0)
        tsum_p = jnp.where(mask, 1.0, 0.0).reshape(_HT // 8, 8, 512).sum(axis=0)
        nom_acc[c] += nom_p
        isum_acc[c] += isum_p
        tsum_acc[c] += tsum_p

    @pl.when(s == _NS - 1)
    def _finish():
        nom = jnp.sum(nom_acc[...], axis=(1, 2))
        isum = jnp.sum(isum_acc[...], axis=(1, 2))
        tsum = jnp.sum(tsum_acc[...], axis=(1, 2))
        frac = (2.0 * nom + _SMOOTH) / (isum + tsum + _SMOOTH)
        loss = -jnp.sum(frac) / _NC
        out_ref[0, 0, :] = jnp.full((128,), loss, dtype=jnp.float32)


def _tc_losses(predictions, targets):
    out = pl.pallas_call(
        _dice_tc_body,
        grid=(_NB_TC, _NS),
        in_specs=[
            pl.BlockSpec((1, _NC, _HT, 512), lambda b, s: (b, 0, s, 0)),
            pl.BlockSpec((1, _HT, 512), lambda b, s: (b, s, 0)),
        ],
        out_specs=pl.BlockSpec((1, 1, 128), lambda b, s: (b, 0, 0)),
        out_shape=jax.ShapeDtypeStruct((_NB_TC, 1, 128), jnp.float32),
        scratch_shapes=[
            pltpu.VMEM((_NC, 8, 512), jnp.float32),
            pltpu.VMEM((_NC, 8, 512), jnp.float32),
            pltpu.VMEM((_NC, 8, 512), jnp.float32),
        ],
    )(predictions, targets)
    return out[:, 0, 0]


def _dice_sc_body(pred_hbm, tgt_hbm, out_hbm,
                  tgt_v, pb0, pb1, res_v, semt, sem0, sem1, semo):
    wid = lax.axis_index("s") * 2 + lax.axis_index("c")
    r0 = wid * _ROWS

    tw0 = pltpu.async_copy(tgt_hbm.at[_SC_B0, pl.ds(r0, _ROWS), :],
                           tgt_v.at[pl.ds(0, _ROWS), :], semt)
    tw1 = pltpu.async_copy(tgt_hbm.at[_SC_B0 + 1, pl.ds(r0, _ROWS), :],
                           tgt_v.at[pl.ds(_ROWS, _ROWS), :], semt)

    def issue(j):
        b, c = _PAIRS[j]
        buf, sem = (pb0, sem0) if j % 2 == 0 else (pb1, sem1)
        return pltpu.async_copy(
            pred_hbm.at[b, c, pl.ds(r0, _ROWS), :], buf, sem)

    handles = [issue(0)]
    tw0.wait()
    tw1.wait()

    zero = jnp.zeros((16,), jnp.float32)
    for j, (b, c) in enumerate(_PAIRS):
        if j + 1 < _NPAIR:
            handles.append(issue(j + 1))
        handles[j].wait()
        buf = pb0 if j % 2 == 0 else pb1
        ti = b - _SC_B0

        def vec_body(v, carry, buf=buf, ti=ti, c=c):
            nom_v, isum_v, tsum_v = carry
            r = v >> 5
            l = (v & 31) * 16
            p = buf[r, pl.ds(l, 16)]
            t = tgt_v[ti * _ROWS + r, pl.ds(l, 16)]
            maskf = jnp.where(t == c, 1.0, 0.0)
            return (nom_v + p * maskf, isum_v + p, tsum_v + maskf)

        nom_v, isum_v, tsum_v = lax.fori_loop(
            0, _CHUNK // 16, vec_body, (zero, zero, zero), unroll=8)
        res_v[pl.ds((0 * _RES_SLOTS + j) * 16, 16)] = nom_v
        res_v[pl.ds((1 * _RES_SLOTS + j) * 16, 16)] = isum_v
        res_v[pl.ds((2 * _RES_SLOTS + j) * 16, 16)] = tsum_v

    pltpu.async_copy(res_v, out_hbm.at[pl.ds(wid * _RES_LEN, _RES_LEN)],
                     semo).wait()


@functools.partial(
    pl.kernel,
    out_type=jax.ShapeDtypeStruct((_NW * _RES_LEN,), jnp.float32),
    mesh=plsc.VectorSubcoreMesh(core_axis_name="c", subcore_axis_name="s"),
    scratch_types=[
        pltpu.VMEM((_NB_SC * _ROWS, 512), jnp.int32),
        pltpu.VMEM((_ROWS, 512), jnp.float32),
        pltpu.VMEM((_ROWS, 512), jnp.float32),
        pltpu.VMEM((_RES_LEN,), jnp.float32),
        pltpu.SemaphoreType.DMA,
        pltpu.SemaphoreType.DMA,
        pltpu.SemaphoreType.DMA,
        pltpu.SemaphoreType.DMA,
    ],
)
def _sc_partials(pred_hbm, tgt_hbm, out_hbm, *scratch):
    _dice_sc_body(pred_hbm, tgt_hbm, out_hbm, *scratch)


def kernel(predictions, targets):
    loss_tc = _tc_losses(predictions, targets)
    parts = _sc_partials(predictions, targets)
    parts = parts.reshape(_NW, 3, _RES_SLOTS, 16)

    sums = parts.sum(axis=(0, 3))
    nom = sums[0, :_NPAIR].reshape(_NB_SC, _NC)
    isum = sums[1, :_NPAIR].reshape(_NB_SC, _NC)
    tsum = sums[2, :_NPAIR].reshape(_NB_SC, _NC)
    frac = (2.0 * nom + _SMOOTH) / (isum + tsum + _SMOOTH)
    loss_sc = -jnp.mean(frac, axis=1)

    return jnp.concatenate([loss_tc, loss_sc])


# speedup vs baseline: 2.4668x; 1.0033x over previous
import functools

import jax
import jax.numpy as jnp
from jax import lax
from jax.experimental import pallas as pl
from jax.experimental.pallas import tpu as pltpu
from jax.experimental.pallas import tpu_sc as plsc

_SMOOTH = 1.0
_NC = 19
_HT = 256
_NS = 512 // _HT
_NB_TC = 6
_SC_B0 = _NB_TC
_NB_SC = 8 - _NB_TC

_P = 512 * 512
_NW = 32
_ROWS = 512 // _NW
_CHUNK = _ROWS * 512
_PAIRS = [(b, c) for b in range(_SC_B0, 8) for c in range(_NC)]
_NPAIR = len(_PAIRS)
_RES_SLOTS = 40
_RES_LEN = 3 * _RES_SLOTS * 16


def _dice_tc_body(pred_ref, tgt_ref, out_ref, nom_acc, isum_acc, tsum_acc):
    s = pl.program_id(1)

    @pl.when(s == 0)
    def _init():
        nom_acc[...] = jnp.zeros_like(nom_acc)
        isum_acc[...] = jnp.zeros_like(isum_acc)
        tsum_acc[...] = jnp.zeros_like(tsum_acc)

    tgt = tgt_ref[0]
    for c in range(_NC):
        pred_c = pred_ref[0, c]
        mask = tgt == c
        nom_p = jnp.where(mask, pred_c, 0.0).reshape(_HT // 8, 8, 512).sum(axis=0)
        isum_p = pred_c.reshape(_HT // 8, 8, 512).sum(axis=0)
        tsum_p = jnp.where(mask, 1.0, 0.0).reshape(_HT // 8, 8, 512).sum(axis=0)
        nom_acc[c] += nom_p
        isum_acc[c] += isum_p
        tsum_acc[c] += tsum_p

    @pl.when(s == _NS - 1)
    def _finish():
        nom = jnp.sum(nom_acc[...], axis=(1, 2))
        isum = jnp.sum(isum_acc[...], axis=(1, 2))
        tsum = jnp.sum(tsum_acc[...], axis=(1, 2))
        frac = (2.0 * nom + _SMOOTH) / (isum + tsum + _SMOOTH)
        loss = -jnp.sum(frac) / _NC
        out_ref[0, 0, :] = jnp.full((128,), loss, dtype=jnp.float32)


def _tc_losses(predictions, targets):
    out = pl.pallas_call(
        _dice_tc_body,
        grid=(_NB_TC, _NS),
        in_specs=[
            pl.BlockSpec((1, _NC, _HT, 512), lambda b, s: (b, 0, s, 0)),
            pl.BlockSpec((1, _HT, 512), lambda b, s: (b, s, 0)),
        ],
        out_specs=pl.BlockSpec((1, 1, 128), lambda b, s: (b, 0, 0)),
        out_shape=jax.ShapeDtypeStruct((_NB_TC, 1, 128), jnp.float32),
        scratch_shapes=[
            pltpu.VMEM((_NC, 8, 512), jnp.float32),
            pltpu.VMEM((_NC, 8, 512), jnp.float32),
            pltpu.VMEM((_NC, 8, 512), jnp.float32),
        ],
    )(predictions, targets)
    return out[:, 0, 0]


def _dice_sc_body(pred_hbm, tgt_hbm, out_hbm,
                  tgt_v, pb0, pb1, res_v, semt, sem0, sem1, semo):
    wid = lax.axis_index("s") * 2 + lax.axis_index("c")
    r0 = wid * _ROWS

    tw0 = pltpu.async_copy(tgt_hbm.at[_SC_B0, pl.ds(r0, _ROWS), :],
                           tgt_v.at[pl.ds(0, _ROWS), :], semt)
    tw1 = pltpu.async_copy(tgt_hbm.at[_SC_B0 + 1, pl.ds(r0, _ROWS), :],
                           tgt_v.at[pl.ds(_ROWS, _ROWS), :], semt)

    def issue(j):
        b, c = _PAIRS[j]
        buf, sem = (pb0, sem0) if j % 2 == 0 else (pb1, sem1)
        return pltpu.async_copy(
            pred_hbm.at[b, c, pl.ds(r0, _ROWS), :], buf, sem)

    handles = [issue(0)]
    tw0.wait()
    tw1.wait()

    zero = jnp.zeros((16,), jnp.float32)
    for j, (b, c) in enumerate(_PAIRS):
        if j + 1 < _NPAIR:
            handles.append(issue(j + 1))
        handles[j].wait()
        buf = pb0 if j % 2 == 0 else pb1
        ti = b - _SC_B0

        def vec_body(v, carry, buf=buf, ti=ti, c=c):
            nom_a, isum_a, tsum_a, nom_b, isum_b, tsum_b = carry
            r = v >> 4
            la = (v & 15) * 32
            pa = buf[r, pl.ds(la, 16)]
            ta = tgt_v[ti * _ROWS + r, pl.ds(la, 16)]
            pb = buf[r, pl.ds(la + 16, 16)]
            tb = tgt_v[ti * _ROWS + r, pl.ds(la + 16, 16)]
            ma = jnp.where(ta == c, 1.0, 0.0)
            mb = jnp.where(tb == c, 1.0, 0.0)
            return (nom_a + pa * ma, isum_a + pa, tsum_a + ma,
                    nom_b + pb * mb, isum_b + pb, tsum_b + mb)

        nom_a, isum_a, tsum_a, nom_b, isum_b, tsum_b = lax.fori_loop(
            0, _CHUNK // 32, vec_body, (zero,) * 6, unroll=8)
        nom_v = nom_a + nom_b
        isum_v = isum_a + isum_b
        tsum_v = tsum_a + tsum_b
        res_v[pl.ds((0 * _RES_SLOTS + j) * 16, 16)] = nom_v
        res_v[pl.ds((1 * _RES_SLOTS + j) * 16, 16)] = isum_v
        res_v[pl.ds((2 * _RES_SLOTS + j) * 16, 16)] = tsum_v

    pltpu.async_copy(res_v, out_hbm.at[pl.ds(wid * _RES_LEN, _RES_LEN)],
                     semo).wait()


@functools.partial(
    pl.kernel,
    out_type=jax.ShapeDtypeStruct((_NW * _RES_LEN,), jnp.float32),
    mesh=plsc.VectorSubcoreMesh(core_axis_name="c", subcore_axis_name="s"),
    scratch_types=[
        pltpu.VMEM((_NB_SC * _ROWS, 512), jnp.int32),
        pltpu.VMEM((_ROWS, 512), jnp.float32),
        pltpu.VMEM((_ROWS, 512), jnp.float32),
        pltpu.VMEM((_RES_LEN,), jnp.float32),
        pltpu.SemaphoreType.DMA,
        pltpu.SemaphoreType.DMA,
        pltpu.SemaphoreType.DMA,
        pltpu.SemaphoreType.DMA,
    ],
)
def _sc_partials(pred_hbm, tgt_hbm, out_hbm, *scratch):
    _dice_sc_body(pred_hbm, tgt_hbm, out_hbm, *scratch)


def kernel(predictions, targets):
    parts = _sc_partials(predictions, targets)
    loss_tc = _tc_losses(predictions, targets)
    parts = parts.reshape(_NW, 3, _RES_SLOTS, 16)

    sums = parts.sum(axis=(0, 3))
    nom = sums[0, :_NPAIR].reshape(_NB_SC, _NC)
    isum = sums[1, :_NPAIR].reshape(_NB_SC, _NC)
    tsum = sums[2, :_NPAIR].reshape(_NB_SC, _NC)
    frac = (2.0 * nom + _SMOOTH) / (isum + tsum + _SMOOTH)
    loss_sc = -jnp.mean(frac, axis=1)

    return jnp.concatenate([loss_tc, loss_sc])
